# Initial kernel scaffold; baseline (speedup 1.0000x reference)
import jax
import jax.numpy as jnp
from jax.experimental import pallas as pl


def kernel(x, edge_index, batch, W1, a_src1, a_dst1, b1, W2, a_src2, a_dst2, b2, fc1_w, fc1_b, fc2_w, fc2_b):
    raise NotImplementedError("write your pallas kernel here")


# speedup vs baseline: 66.9616x; 66.9616x over previous
import functools

import jax
import jax.numpy as jnp
import numpy as np
from jax import lax
from jax.experimental import pallas as pl
from jax.experimental.pallas import tpu as pltpu
from jax.experimental.pallas import tpu_sc as plsc

N = 10000
DIN = 128
HID = 8
H1 = 8
NG = 64
NC = 10

NW = 32
C = 128
CH = 80
EPW = C * CH
EP = NW * EPW
NPAD = N + 128

_f32 = jnp.float32
_i32 = jnp.int32


def _dyngather(v, idx):
    dnums = lax.GatherDimensionNumbers(
        offset_dims=(), collapsed_slice_dims=(0,), start_index_map=(0,))
    return lax.gather(v, idx[:, None], dnums, (1,),
                      mode=lax.GatherScatterMode.PROMISE_IN_BOUNDS)


def _tc_pre1(x_ref, w1_ref, ms_ref, md_ref, h_ref, as_ref, ad_ref, ws_ref):
    h = jnp.dot(x_ref[...], w1_ref[...], preferred_element_type=_f32)
    h_ref[...] = h
    a_s = jnp.dot(h, ms_ref[...], preferred_element_type=_f32)
    a_d = jnp.dot(h, md_ref[...], preferred_element_type=_f32)
    as_ref[...] = a_s
    ad_ref[...] = a_d
    s = a_s + a_d
    s = jnp.maximum(s, 0.2 * s)
    ws_ref[...] = jnp.exp(s)


def _run_pre1(x, W1, Msrc, Mdst):
    B = 2000
    grid = (N // B,)
    return pl.pallas_call(
        _tc_pre1,
        grid=grid,
        in_specs=[
            pl.BlockSpec((B, DIN), lambda i: (i, 0)),
            pl.BlockSpec((DIN, H1 * HID), lambda i: (0, 0)),
            pl.BlockSpec((H1 * HID, H1), lambda i: (0, 0)),
            pl.BlockSpec((H1 * HID, H1), lambda i: (0, 0)),
        ],
        out_specs=[
            pl.BlockSpec((B, H1 * HID), lambda i: (i, 0)),
            pl.BlockSpec((B, H1), lambda i: (i, 0)),
            pl.BlockSpec((B, H1), lambda i: (i, 0)),
            pl.BlockSpec((B, H1), lambda i: (i, 0)),
        ],
        out_shape=[
            jax.ShapeDtypeStruct((N, H1 * HID), _f32),
            jax.ShapeDtypeStruct((N, H1), _f32),
            jax.ShapeDtypeStruct((N, H1), _f32),
            jax.ShapeDtypeStruct((N, H1), _f32),
        ],
    )(x, W1, Msrc, Mdst)


def _sc1_body(h_hbm, as_hbm, ad_hbm, src_hbm, dst_hbm, zz_hbm, acc_hbm,
              h_sp, as_sp, ad_sp, acc_sp,
              src_t, dst_t, as_g, ad_g, h_g, buf):
    cid = lax.axis_index("c")
    sid = lax.axis_index("s")
    wid = cid * 16 + sid

    s6 = jnp.minimum(sid * 632, N - 632)
    s3 = jnp.minimum(sid * 640, NPAD - 640)
    pltpu.sync_copy(h_hbm.at[pl.ds(s6, 632)], h_sp.at[pl.ds(s6, 632)])
    pltpu.sync_copy(as_hbm.at[pl.ds(s6, 632)], as_sp.at[pl.ds(s6, 632)])
    pltpu.sync_copy(ad_hbm.at[pl.ds(s6, 632)], ad_sp.at[pl.ds(s6, 632)])
    pltpu.sync_copy(zz_hbm.at[pl.ds(s3, 640)], acc_sp.at[pl.ds(s3, 640)])
    plsc.subcore_barrier()

    lane = lax.iota(_i32, 16)
    ilo8 = lane // 8
    col8 = lane % 8
    idxc = [jnp.where(lane < 8, 2 * j, 2 * j + 1).astype(_i32)
            for j in range(8)]

    def chunk_body(j, carry):
        pltpu.sync_copy(src_hbm.at[pl.ds(wid * CH + j, 1)], src_t)
        pltpu.sync_copy(dst_hbm.at[pl.ds(wid * CH + j, 1)], dst_t)
        src_row = src_t.at[0]
        dst_row = dst_t.at[0]
        pltpu.sync_copy(as_sp.at[src_row], as_g)
        pltpu.sync_copy(ad_sp.at[dst_row], ad_g)
        pltpu.sync_copy(h_sp.at[src_row], h_g)

        def pair_body(p, c2):
            rows2 = 2 * p + ilo8
            a = plsc.load_gather(as_g, [rows2, col8])
            b = plsc.load_gather(ad_g, [rows2, col8])
            s = a + b
            s = jnp.maximum(s, 0.2 * s)
            w16 = jnp.exp(s)
            plsc.store_scatter(buf, [rows2, 64 + col8], w16)
            for j8 in range(8):
                e_rel = 2 * p + (1 if j8 >= 4 else 0)
                er = jnp.broadcast_to(e_rel, (16,)).astype(_i32)
                cols = (j8 % 4) * 16 + lane
                hv = plsc.load_gather(h_g, [er, cols])
                wb = _dyngather(w16, idxc[j8])
                plsc.store_scatter(buf, [er, cols], wb * hv)
            return c2

        lax.fori_loop(0, 64, pair_body, 0)
        pltpu.sync_copy(buf, acc_sp.at[dst_row], add=True)
        return carry

    lax.fori_loop(0, CH, chunk_body, 0)
    plsc.subcore_barrier()
    pltpu.sync_copy(acc_sp.at[pl.ds(s3, 640)],
                    acc_hbm.at[cid].at[pl.ds(s3, 640)])


def _run_sc1(h, asrc, adst, srcp, dstp, zz):
    mesh = plsc.VectorSubcoreMesh(core_axis_name="c", subcore_axis_name="s", num_cores=2, num_subcores=16)
    f = pl.kernel(
        _sc1_body,
        out_type=[jax.ShapeDtypeStruct((2, NPAD, 72), _f32)],
        mesh=mesh,
        compiler_params=pltpu.CompilerParams(needs_layout_passes=False, use_tc_tiling_on_sc=False),
        scratch_types=[
            pltpu.MemorySpace.VMEM_SHARED((N, 64), _f32),
            pltpu.MemorySpace.VMEM_SHARED((N, 8), _f32),
            pltpu.MemorySpace.VMEM_SHARED((N, 8), _f32),
            pltpu.MemorySpace.VMEM_SHARED((NPAD, 72), _f32),
            pltpu.MemorySpace.VMEM((1, C), _i32),
            pltpu.MemorySpace.VMEM((1, C), _i32),
            pltpu.MemorySpace.VMEM((C, 8), _f32),
            pltpu.MemorySpace.VMEM((C, 8), _f32),
            pltpu.MemorySpace.VMEM((C, 64), _f32),
            pltpu.MemorySpace.VMEM((C, 72), _f32),
        ],
    )
    return f(h, asrc, adst, srcp, dstp, zz)[0]


def _tc_merge1(a0_ref, a1_ref, ws_ref, h_ref, e8_ref, b1_ref, w2_ref,
               ms2_ref, md2_ref, h2_ref, as2_ref, ad2_ref, ws2_ref):
    a0 = a0_ref[...]
    a1 = a1_ref[...]
    ws = ws_ref[...]
    h = h_ref[...]
    e8 = e8_ref[...]
    out_t = a0[:, :64] + a1[:, :64] + jnp.dot(ws, e8, preferred_element_type=_f32) * h
    ssum = a0[:, 64:72] + a1[:, 64:72] + ws
    inv = 1.0 / (ssum + 1e-16)
    x2 = out_t * jnp.dot(inv, e8, preferred_element_type=_f32) + b1_ref[...]
    x2 = jnp.where(x2 > 0, x2, jnp.exp(x2) - 1.0)
    h2 = jnp.dot(x2, w2_ref[...], preferred_element_type=_f32)
    h2_ref[...] = h2
    a_s = jnp.dot(h2, ms2_ref[...], preferred_element_type=_f32)
    a_d = jnp.dot(h2, md2_ref[...], preferred_element_type=_f32)
    as2_ref[...] = a_s
    ad2_ref[...] = a_d
    s = a_s + a_d
    s = jnp.maximum(s, 0.2 * s)
    ws2_ref[...] = jnp.exp(s)


def _run_merge1(acc0, acc1, wself, h, E8, b1r, W2, ms2, md2):
    B = 2000
    grid = (N // B,)
    return pl.pallas_call(
        _tc_merge1,
        grid=grid,
        in_specs=[
            pl.BlockSpec((B, 72), lambda i: (i, 0)),
            pl.BlockSpec((B, 72), lambda i: (i, 0)),
            pl.BlockSpec((B, H1), lambda i: (i, 0)),
            pl.BlockSpec((B, 64), lambda i: (i, 0)),
            pl.BlockSpec((H1, 64), lambda i: (0, 0)),
            pl.BlockSpec((1, 64), lambda i: (0, 0)),
            pl.BlockSpec((64, HID), lambda i: (0, 0)),
            pl.BlockSpec((HID, 1), lambda i: (0, 0)),
            pl.BlockSpec((HID, 1), lambda i: (0, 0)),
        ],
        out_specs=[
            pl.BlockSpec((B, HID), lambda i: (i, 0)),
            pl.BlockSpec((B, 1), lambda i: (i, 0)),
            pl.BlockSpec((B, 1), lambda i: (i, 0)),
            pl.BlockSpec((B, 1), lambda i: (i, 0)),
        ],
        out_shape=[
            jax.ShapeDtypeStruct((N, HID), _f32),
            jax.ShapeDtypeStruct((N, 1), _f32),
            jax.ShapeDtypeStruct((N, 1), _f32),
            jax.ShapeDtypeStruct((N, 1), _f32),
        ],
    )(acc0, acc1, wself, h, E8, b1r, W2, ms2, md2)


def _sc2_body(h2_hbm, t2_hbm, src_hbm, dst_hbm, zz_hbm, acc_hbm,
              h2_sp, t2_sp, acc_sp,
              src_t, dst_t, ta_g, tb_g, h2_g, buf):
    cid = lax.axis_index("c")
    sid = lax.axis_index("s")
    wid = cid * 16 + sid

    s6 = jnp.minimum(sid * 632, N - 632)
    s3 = jnp.minimum(sid * 640, NPAD - 640)
    pltpu.sync_copy(h2_hbm.at[pl.ds(s6, 632)], h2_sp.at[pl.ds(s6, 632)])
    pltpu.sync_copy(t2_hbm.at[pl.ds(s6, 632)], t2_sp.at[pl.ds(s6, 632)])
    pltpu.sync_copy(zz_hbm.at[pl.ds(s3, 640)], acc_sp.at[pl.ds(s3, 640)])
    plsc.subcore_barrier()

    lane = lax.iota(_i32, 16)
    col8 = lane % 8
    zz16 = jnp.zeros((16,), _i32)
    on16 = jnp.ones((16,), _i32)
    idxb = [jnp.full((16,), j, _i32) for j in range(16)]

    def chunk_body(j, carry):
        pltpu.sync_copy(src_hbm.at[pl.ds(wid * CH + j, 1)], src_t)
        pltpu.sync_copy(dst_hbm.at[pl.ds(wid * CH + j, 1)], dst_t)
        src_row = src_t.at[0]
        dst_row = dst_t.at[0]
        pltpu.sync_copy(t2_sp.at[src_row], ta_g)
        pltpu.sync_copy(t2_sp.at[dst_row], tb_g)
        pltpu.sync_copy(h2_sp.at[src_row], h2_g)

        def grp_body(g, c2):
            rows16 = g * 16 + lane
            a = plsc.load_gather(ta_g, [rows16, zz16])
            b = plsc.load_gather(tb_g, [rows16, on16])
            s = a + b
            s = jnp.maximum(s, 0.2 * s)
            w16 = jnp.exp(s)
            for j16 in range(16):
                e_rel = g * 16 + j16
                er = jnp.broadcast_to(e_rel, (16,)).astype(_i32)
                hv = plsc.load_gather(h2_g, [er, col8])
                wb = _dyngather(w16, idxb[j16])
                m = jnp.where(lane < 8, wb * hv,
                              jnp.where(lane == 8, wb, 0.0))
                plsc.store_scatter(buf, [er, lane], m)
            return c2

        lax.fori_loop(0, 8, grp_body, 0)
        pltpu.sync_copy(buf, acc_sp.at[dst_row], add=True)
        return carry

    lax.fori_loop(0, CH, chunk_body, 0)
    plsc.subcore_barrier()
    pltpu.sync_copy(acc_sp.at[pl.ds(s3, 640)],
                    acc_hbm.at[cid].at[pl.ds(s3, 640)])


def _run_sc2(h2, t2, srcp, dstp, zz):
    mesh = plsc.VectorSubcoreMesh(core_axis_name="c", subcore_axis_name="s", num_cores=2, num_subcores=16)
    f = pl.kernel(
        _sc2_body,
        out_type=[jax.ShapeDtypeStruct((2, NPAD, 16), _f32)],
        mesh=mesh,
        compiler_params=pltpu.CompilerParams(needs_layout_passes=False, use_tc_tiling_on_sc=False),
        scratch_types=[
            pltpu.MemorySpace.VMEM_SHARED((N, HID), _f32),
            pltpu.MemorySpace.VMEM_SHARED((N, 2), _f32),
            pltpu.MemorySpace.VMEM_SHARED((NPAD, 16), _f32),
            pltpu.MemorySpace.VMEM((1, C), _i32),
            pltpu.MemorySpace.VMEM((1, C), _i32),
            pltpu.MemorySpace.VMEM((C, 2), _f32),
            pltpu.MemorySpace.VMEM((C, 2), _f32),
            pltpu.MemorySpace.VMEM((C, HID), _f32),
            pltpu.MemorySpace.VMEM((C, 16), _f32),
        ],
    )
    return f(h2, t2, srcp, dstp, zz)[0]


def _tc_final(a0_ref, a1_ref, ws2_ref, h2_ref, b2_ref, bat_ref,
              f1w_ref, f1b_ref, f2w_ref, f2b_ref, out_ref, accg, acccnt):
    i = pl.program_id(0)
    nsteps = pl.num_programs(0)
    a0 = a0_ref[...]
    a1 = a1_ref[...]
    ws2 = ws2_ref[...]
    num = a0[:, :8] + a1[:, :8] + ws2 * h2_ref[...]
    den = a0[:, 8:9] + a1[:, 8:9] + ws2
    h2f = num / (den + 1e-16) + b2_ref[...]
    bat = bat_ref[0]
    gid = lax.broadcasted_iota(_i32, (NG, 1), 0)
    oh = (bat == gid).astype(_f32)
    part = jnp.dot(oh, h2f, preferred_element_type=_f32)
    cnt = jnp.sum(oh, axis=1, keepdims=True)

    @pl.when(i == 0)
    def _():
        accg[...] = part
        acccnt[...] = cnt

    @pl.when(i > 0)
    def _():
        accg[...] += part
        acccnt[...] += cnt

    @pl.when(i == nsteps - 1)
    def _():
        g = accg[...] / jnp.maximum(acccnt[...], 1.0)
        z = jnp.maximum(jnp.dot(g, f1w_ref[...], preferred_element_type=_f32)
                        + f1b_ref[...], 0.0)
        z2 = jnp.dot(z, f2w_ref[...], preferred_element_type=_f32) + f2b_ref[...]
        mx = jnp.max(z2, axis=1, keepdims=True)
        lse = mx + jnp.log(jnp.sum(jnp.exp(z2 - mx), axis=1, keepdims=True))
        out_ref[...] = z2 - lse


def _run_final(acc0, acc1, wself2, h2, b2r, bat2d, fc1_w, fc1b, fc2_w, fc2b):
    B = 2000
    grid = (N // B,)
    return pl.pallas_call(
        _tc_final,
        grid=grid,
        in_specs=[
            pl.BlockSpec((B, 16), lambda i: (i, 0)),
            pl.BlockSpec((B, 16), lambda i: (i, 0)),
            pl.BlockSpec((B, 1), lambda i: (i, 0)),
            pl.BlockSpec((B, HID), lambda i: (i, 0)),
            pl.BlockSpec((1, HID), lambda i: (0, 0)),
            pl.BlockSpec((1, 1, B), lambda i: (i, 0, 0)),
            pl.BlockSpec((HID, 20), lambda i: (0, 0)),
            pl.BlockSpec((1, 20), lambda i: (0, 0)),
            pl.BlockSpec((20, NC), lambda i: (0, 0)),
            pl.BlockSpec((1, NC), lambda i: (0, 0)),
        ],
        out_specs=pl.BlockSpec((NG, NC), lambda i: (0, 0)),
        out_shape=jax.ShapeDtypeStruct((NG, NC), _f32),
        scratch_shapes=[
            pltpu.MemorySpace.VMEM((NG, HID), _f32),
            pltpu.MemorySpace.VMEM((NG, 1), _f32),
        ],
    )(acc0, acc1, wself2, h2, b2r, bat2d, fc1_w, fc1b, fc2_w, fc2b)


def kernel(x, edge_index, batch, W1, a_src1, a_dst1, b1, W2, a_src2, a_dst2,
           b2, fc1_w, fc1_b, fc2_w, fc2_b):
    E = edge_index.shape[1]
    npd = EP - E
    src_p = jnp.concatenate([edge_index[0], jnp.zeros((npd,), _i32)])
    dst_p = jnp.concatenate(
        [edge_index[1], N + (jnp.arange(npd, dtype=_i32) % 128)])
    srcp = src_p.reshape(NW * CH, C)
    dstp = dst_p.reshape(NW * CH, C)

    rows = np.arange(H1 * HID)
    Msrc = jnp.zeros((H1 * HID, H1), _f32).at[rows, rows // HID].set(
        a_src1.reshape(-1))
    Mdst = jnp.zeros((H1 * HID, H1), _f32).at[rows, rows // HID].set(
        a_dst1.reshape(-1))
    E8 = jnp.asarray((np.arange(64)[None, :] // 8 == np.arange(8)[:, None])
                     .astype(np.float32))
    zz1 = jnp.zeros((NPAD, 72), _f32)
    zz2 = jnp.zeros((NPAD, 16), _f32)

    h, asrc, adst, wself = _run_pre1(x, W1, Msrc, Mdst)
    accp = _run_sc1(h, asrc, adst, srcp, dstp, zz1)
    h2, asrc2, adst2, wself2 = _run_merge1(
        accp[0, :N], accp[1, :N], wself, h, E8, b1.reshape(1, 64), W2,
        a_src2.reshape(HID, 1), a_dst2.reshape(HID, 1))
    t2 = jnp.concatenate([asrc2, adst2], axis=1)
    accp2 = _run_sc2(h2, t2, srcp, dstp, zz2)
    out = _run_final(accp2[0, :N], accp2[1, :N], wself2, h2,
                     b2.reshape(1, HID), batch.reshape(N // 2000, 1, 2000),
                     fc1_w, fc1_b.reshape(1, 20), fc2_w, fc2_b.reshape(1, NC))
    return out


# speedup vs baseline: 101.2655x; 1.5123x over previous
import functools

import jax
import jax.numpy as jnp
import numpy as np
from jax import lax
from jax.experimental import pallas as pl
from jax.experimental.pallas import tpu as pltpu
from jax.experimental.pallas import tpu_sc as plsc

N = 10000
DIN = 128
HID = 8
H1 = 8
NG = 64
NC = 10

NW = 32
C = 80
CH = 128
EPW = C * CH
EP = NW * EPW
NPAD = N + 128

_f32 = jnp.float32
_i32 = jnp.int32


def _dyngather(v, idx):
    dnums = lax.GatherDimensionNumbers(
        offset_dims=(), collapsed_slice_dims=(0,), start_index_map=(0,))
    return lax.gather(v, idx[:, None], dnums, (1,),
                      mode=lax.GatherScatterMode.PROMISE_IN_BOUNDS)


def _tc_pre1(x_ref, w1_ref, ms_ref, md_ref, h_ref, as_ref, ad_ref, ws_ref):
    h = jnp.dot(x_ref[...], w1_ref[...], preferred_element_type=_f32)
    h_ref[...] = h
    a_s = jnp.dot(h, ms_ref[...], preferred_element_type=_f32)
    a_d = jnp.dot(h, md_ref[...], preferred_element_type=_f32)
    as_ref[...] = a_s
    ad_ref[...] = a_d
    s = a_s + a_d
    s = jnp.maximum(s, 0.2 * s)
    ws_ref[...] = jnp.exp(s)


def _run_pre1(x, W1, Msrc, Mdst):
    B = 2000
    grid = (N // B,)
    return pl.pallas_call(
        _tc_pre1,
        grid=grid,
        in_specs=[
            pl.BlockSpec((B, DIN), lambda i: (i, 0)),
            pl.BlockSpec((DIN, H1 * HID), lambda i: (0, 0)),
            pl.BlockSpec((H1 * HID, H1), lambda i: (0, 0)),
            pl.BlockSpec((H1 * HID, H1), lambda i: (0, 0)),
        ],
        out_specs=[
            pl.BlockSpec((B, H1 * HID), lambda i: (i, 0)),
            pl.BlockSpec((B, H1), lambda i: (i, 0)),
            pl.BlockSpec((B, H1), lambda i: (i, 0)),
            pl.BlockSpec((B, H1), lambda i: (i, 0)),
        ],
        out_shape=[
            jax.ShapeDtypeStruct((N, H1 * HID), _f32),
            jax.ShapeDtypeStruct((N, H1), _f32),
            jax.ShapeDtypeStruct((N, H1), _f32),
            jax.ShapeDtypeStruct((N, H1), _f32),
        ],
    )(x, W1, Msrc, Mdst)


def _sc1_body(h_hbm, as_hbm, ad_hbm, src_hbm, dst_hbm, zz_hbm, acc_hbm,
              h_sp, as_sp, ad_sp, acc_sp,
              src_t, dst_t, as_g, ad_g, h_g, buf, sem_i, sem_g, sem_s):
    cid = lax.axis_index("c")
    sid = lax.axis_index("s")
    wid = cid * 16 + sid

    s6 = jnp.minimum(sid * 632, N - 632)
    s3 = jnp.minimum(sid * 640, NPAD - 640)
    pltpu.sync_copy(h_hbm.at[pl.ds(s6, 632)], h_sp.at[pl.ds(s6, 632)])
    pltpu.sync_copy(as_hbm.at[pl.ds(s6, 632)], as_sp.at[pl.ds(s6, 632)])
    pltpu.sync_copy(ad_hbm.at[pl.ds(s6, 632)], ad_sp.at[pl.ds(s6, 632)])
    pltpu.sync_copy(zz_hbm.at[pl.ds(s3, 640)], acc_sp.at[pl.ds(s3, 640)])
    plsc.subcore_barrier()

    lane = lax.iota(_i32, 16)
    ilo8 = lane // 8
    col8 = lane % 8
    idxc = [jnp.where(lane < 8, 2 * j, 2 * j + 1).astype(_i32)
            for j in range(8)]
    base = wid * CH

    def idx_issue(j, s4):
        pltpu.async_copy(src_hbm.at[pl.ds(base + j, 1)],
                         src_t.at[pl.ds(s4, 1)], sem_i.at[s4])
        pltpu.async_copy(dst_hbm.at[pl.ds(base + j, 1)],
                         dst_t.at[pl.ds(s4, 1)], sem_i.at[s4])

    def idx_wait(s4):
        pltpu.make_async_copy(src_hbm.at[pl.ds(0, 1)],
                              src_t.at[pl.ds(s4, 1)], sem_i.at[s4]).wait()
        pltpu.make_async_copy(dst_hbm.at[pl.ds(0, 1)],
                              dst_t.at[pl.ds(s4, 1)], sem_i.at[s4]).wait()

    def gat_issue(s4, s2):
        src_row = src_t.at[s4]
        dst_row = dst_t.at[s4]
        pltpu.async_copy(as_sp.at[src_row], as_g.at[s2], sem_g.at[s2])
        pltpu.async_copy(ad_sp.at[dst_row], ad_g.at[s2], sem_g.at[s2])
        pltpu.async_copy(h_sp.at[src_row], h_g.at[s2], sem_g.at[s2])

    def gat_wait(s4, s2):
        src_row = src_t.at[s4]
        dst_row = dst_t.at[s4]
        pltpu.make_async_copy(as_sp.at[src_row], as_g.at[s2], sem_g.at[s2]).wait()
        pltpu.make_async_copy(ad_sp.at[dst_row], ad_g.at[s2], sem_g.at[s2]).wait()
        pltpu.make_async_copy(h_sp.at[src_row], h_g.at[s2], sem_g.at[s2]).wait()

    def sc_issue(s4, s2):
        pltpu.async_copy(buf.at[s2], acc_sp.at[dst_t.at[s4]], sem_s.at[s2],
                         add=True)

    def sc_wait(s4, s2):
        pltpu.make_async_copy(buf.at[s2], acc_sp.at[dst_t.at[s4]],
                              sem_s.at[s2]).wait()

    idx_issue(0, 0)
    idx_issue(1, 1)
    idx_wait(0)
    gat_issue(0, 0)

    def chunk_body(j, carry):
        s2 = lax.rem(j, 2)
        s2n = 1 - s2
        s4 = lax.rem(j, 4)
        gat_wait(s4, s2)

        @pl.when(j >= 2)
        def _():
            sc_wait(lax.rem(j - 2, 4), s2)

        @pl.when(j < CH - 1)
        def _():
            idx_wait(lax.rem(j + 1, 4))
            gat_issue(lax.rem(j + 1, 4), s2n)

        @pl.when(j < CH - 2)
        def _():
            idx_issue(j + 2, lax.rem(j + 2, 4))

        sl16 = jnp.broadcast_to(s2, (16,)).astype(_i32)

        def pair_body(p, c2):
            rows2 = 2 * p + ilo8
            a = plsc.load_gather(as_g, [sl16, rows2, col8])
            b = plsc.load_gather(ad_g, [sl16, rows2, col8])
            s = a + b
            s = jnp.maximum(s, 0.2 * s)
            w16 = jnp.exp(s)
            plsc.store_scatter(buf, [sl16, rows2, 64 + col8], w16)
            er0 = jnp.broadcast_to(2 * p, (16,)).astype(_i32)
            er1 = er0 + 1
            for j8 in range(8):
                er = er1 if j8 >= 4 else er0
                cols = (j8 % 4) * 16 + lane
                hv = plsc.load_gather(h_g, [sl16, er, cols])
                wb = _dyngather(w16, idxc[j8])
                plsc.store_scatter(buf, [sl16, er, cols], wb * hv)
            return c2

        lax.fori_loop(0, C // 2, pair_body, 0)
        sc_issue(s4, s2)
        return carry

    lax.fori_loop(0, CH, chunk_body, 0)
    sc_wait((CH - 2) % 4, (CH - 2) % 2)
    sc_wait((CH - 1) % 4, (CH - 1) % 2)
    plsc.subcore_barrier()
    pltpu.sync_copy(acc_sp.at[pl.ds(s3, 640)],
                    acc_hbm.at[cid].at[pl.ds(s3, 640)])


def _run_sc1(h, asrc, adst, srcp, dstp, zz):
    mesh = plsc.VectorSubcoreMesh(core_axis_name="c", subcore_axis_name="s", num_cores=2, num_subcores=16)
    f = pl.kernel(
        _sc1_body,
        out_type=[jax.ShapeDtypeStruct((2, NPAD, 72), _f32)],
        mesh=mesh,
        compiler_params=pltpu.CompilerParams(needs_layout_passes=False, use_tc_tiling_on_sc=False),
        scratch_types=[
            pltpu.MemorySpace.VMEM_SHARED((N, 64), _f32),
            pltpu.MemorySpace.VMEM_SHARED((N, 8), _f32),
            pltpu.MemorySpace.VMEM_SHARED((N, 8), _f32),
            pltpu.MemorySpace.VMEM_SHARED((NPAD, 72), _f32),
            pltpu.MemorySpace.VMEM((4, C), _i32),
            pltpu.MemorySpace.VMEM((4, C), _i32),
            pltpu.MemorySpace.VMEM((2, C, 8), _f32),
            pltpu.MemorySpace.VMEM((2, C, 8), _f32),
            pltpu.MemorySpace.VMEM((2, C, 64), _f32),
            pltpu.MemorySpace.VMEM((2, C, 72), _f32),
            pltpu.SemaphoreType.DMA((4,)),
            pltpu.SemaphoreType.DMA((2,)),
            pltpu.SemaphoreType.DMA((2,)),
        ],
    )
    return f(h, asrc, adst, srcp, dstp, zz)[0]


def _tc_merge1(a0_ref, a1_ref, ws_ref, h_ref, e8_ref, b1_ref, w2_ref,
               ms2_ref, md2_ref, h2_ref, as2_ref, ad2_ref, ws2_ref):
    a0 = a0_ref[...]
    a1 = a1_ref[...]
    ws = ws_ref[...]
    h = h_ref[...]
    e8 = e8_ref[...]
    out_t = a0[:, :64] + a1[:, :64] + jnp.dot(ws, e8, preferred_element_type=_f32) * h
    ssum = a0[:, 64:72] + a1[:, 64:72] + ws
    inv = 1.0 / (ssum + 1e-16)
    x2 = out_t * jnp.dot(inv, e8, preferred_element_type=_f32) + b1_ref[...]
    x2 = jnp.where(x2 > 0, x2, jnp.exp(x2) - 1.0)
    h2 = jnp.dot(x2, w2_ref[...], preferred_element_type=_f32)
    h2_ref[...] = h2
    a_s = jnp.dot(h2, ms2_ref[...], preferred_element_type=_f32)
    a_d = jnp.dot(h2, md2_ref[...], preferred_element_type=_f32)
    as2_ref[...] = a_s
    ad2_ref[...] = a_d
    s = a_s + a_d
    s = jnp.maximum(s, 0.2 * s)
    ws2_ref[...] = jnp.exp(s)


def _run_merge1(acc0, acc1, wself, h, E8, b1r, W2, ms2, md2):
    B = 2000
    grid = (N // B,)
    return pl.pallas_call(
        _tc_merge1,
        grid=grid,
        in_specs=[
            pl.BlockSpec((B, 72), lambda i: (i, 0)),
            pl.BlockSpec((B, 72), lambda i: (i, 0)),
            pl.BlockSpec((B, H1), lambda i: (i, 0)),
            pl.BlockSpec((B, 64), lambda i: (i, 0)),
            pl.BlockSpec((H1, 64), lambda i: (0, 0)),
            pl.BlockSpec((1, 64), lambda i: (0, 0)),
            pl.BlockSpec((64, HID), lambda i: (0, 0)),
            pl.BlockSpec((HID, 1), lambda i: (0, 0)),
            pl.BlockSpec((HID, 1), lambda i: (0, 0)),
        ],
        out_specs=[
            pl.BlockSpec((B, HID), lambda i: (i, 0)),
            pl.BlockSpec((B, 1), lambda i: (i, 0)),
            pl.BlockSpec((B, 1), lambda i: (i, 0)),
            pl.BlockSpec((B, 1), lambda i: (i, 0)),
        ],
        out_shape=[
            jax.ShapeDtypeStruct((N, HID), _f32),
            jax.ShapeDtypeStruct((N, 1), _f32),
            jax.ShapeDtypeStruct((N, 1), _f32),
            jax.ShapeDtypeStruct((N, 1), _f32),
        ],
    )(acc0, acc1, wself, h, E8, b1r, W2, ms2, md2)


def _sc2_body(h2_hbm, t2_hbm, src_hbm, dst_hbm, zz_hbm, acc_hbm,
              h2_sp, t2_sp, acc_sp,
              src_t, dst_t, ta_g, tb_g, h2_g, buf, sem_i, sem_g, sem_s):
    cid = lax.axis_index("c")
    sid = lax.axis_index("s")
    wid = cid * 16 + sid

    s6 = jnp.minimum(sid * 632, N - 632)
    s3 = jnp.minimum(sid * 640, NPAD - 640)
    pltpu.sync_copy(h2_hbm.at[pl.ds(s6, 632)], h2_sp.at[pl.ds(s6, 632)])
    pltpu.sync_copy(t2_hbm.at[pl.ds(s6, 632)], t2_sp.at[pl.ds(s6, 632)])
    pltpu.sync_copy(zz_hbm.at[pl.ds(s3, 640)], acc_sp.at[pl.ds(s3, 640)])
    plsc.subcore_barrier()

    lane = lax.iota(_i32, 16)
    col8 = lane % 8
    zz16 = jnp.zeros((16,), _i32)
    on16 = jnp.ones((16,), _i32)
    idxb = [jnp.full((16,), j, _i32) for j in range(16)]
    base = wid * CH

    def idx_issue(j, s4):
        pltpu.async_copy(src_hbm.at[pl.ds(base + j, 1)],
                         src_t.at[pl.ds(s4, 1)], sem_i.at[s4])
        pltpu.async_copy(dst_hbm.at[pl.ds(base + j, 1)],
                         dst_t.at[pl.ds(s4, 1)], sem_i.at[s4])

    def idx_wait(s4):
        pltpu.make_async_copy(src_hbm.at[pl.ds(0, 1)],
                              src_t.at[pl.ds(s4, 1)], sem_i.at[s4]).wait()
        pltpu.make_async_copy(dst_hbm.at[pl.ds(0, 1)],
                              dst_t.at[pl.ds(s4, 1)], sem_i.at[s4]).wait()

    def gat_issue(s4, s2):
        src_row = src_t.at[s4]
        dst_row = dst_t.at[s4]
        pltpu.async_copy(t2_sp.at[src_row], ta_g.at[s2], sem_g.at[s2])
        pltpu.async_copy(t2_sp.at[dst_row], tb_g.at[s2], sem_g.at[s2])
        pltpu.async_copy(h2_sp.at[src_row], h2_g.at[s2], sem_g.at[s2])

    def gat_wait(s4, s2):
        src_row = src_t.at[s4]
        dst_row = dst_t.at[s4]
        pltpu.make_async_copy(t2_sp.at[src_row], ta_g.at[s2], sem_g.at[s2]).wait()
        pltpu.make_async_copy(t2_sp.at[dst_row], tb_g.at[s2], sem_g.at[s2]).wait()
        pltpu.make_async_copy(h2_sp.at[src_row], h2_g.at[s2], sem_g.at[s2]).wait()

    def sc_issue(s4, s2):
        pltpu.async_copy(buf.at[s2], acc_sp.at[dst_t.at[s4]], sem_s.at[s2],
                         add=True)

    def sc_wait(s4, s2):
        pltpu.make_async_copy(buf.at[s2], acc_sp.at[dst_t.at[s4]],
                              sem_s.at[s2]).wait()

    idx_issue(0, 0)
    idx_issue(1, 1)
    idx_wait(0)
    gat_issue(0, 0)

    def chunk_body(j, carry):
        s2 = lax.rem(j, 2)
        s2n = 1 - s2
        s4 = lax.rem(j, 4)
        gat_wait(s4, s2)

        @pl.when(j >= 2)
        def _():
            sc_wait(lax.rem(j - 2, 4), s2)

        @pl.when(j < CH - 1)
        def _():
            idx_wait(lax.rem(j + 1, 4))
            gat_issue(lax.rem(j + 1, 4), s2n)

        @pl.when(j < CH - 2)
        def _():
            idx_issue(j + 2, lax.rem(j + 2, 4))

        sl16 = jnp.broadcast_to(s2, (16,)).astype(_i32)

        def grp_body(g, c2):
            rows16 = g * 16 + lane
            a = plsc.load_gather(ta_g, [sl16, rows16, zz16])
            b = plsc.load_gather(tb_g, [sl16, rows16, on16])
            s = a + b
            s = jnp.maximum(s, 0.2 * s)
            w16 = jnp.exp(s)
            for j16 in range(16):
                e_rel = g * 16 + j16
                er = jnp.broadcast_to(e_rel, (16,)).astype(_i32)
                hv = plsc.load_gather(h2_g, [sl16, er, col8])
                wb = _dyngather(w16, idxb[j16])
                m = jnp.where(lane < 8, wb * hv,
                              jnp.where(lane == 8, wb, 0.0))
                plsc.store_scatter(buf, [sl16, er, lane], m)
            return c2

        lax.fori_loop(0, C // 16, grp_body, 0)
        sc_issue(s4, s2)
        return carry

    lax.fori_loop(0, CH, chunk_body, 0)
    sc_wait((CH - 2) % 4, (CH - 2) % 2)
    sc_wait((CH - 1) % 4, (CH - 1) % 2)
    plsc.subcore_barrier()
    pltpu.sync_copy(acc_sp.at[pl.ds(s3, 640)],
                    acc_hbm.at[cid].at[pl.ds(s3, 640)])


def _run_sc2(h2, t2, srcp, dstp, zz):
    mesh = plsc.VectorSubcoreMesh(core_axis_name="c", subcore_axis_name="s", num_cores=2, num_subcores=16)
    f = pl.kernel(
        _sc2_body,
        out_type=[jax.ShapeDtypeStruct((2, NPAD, 16), _f32)],
        mesh=mesh,
        compiler_params=pltpu.CompilerParams(needs_layout_passes=False, use_tc_tiling_on_sc=False),
        scratch_types=[
            pltpu.MemorySpace.VMEM_SHARED((N, HID), _f32),
            pltpu.MemorySpace.VMEM_SHARED((N, 2), _f32),
            pltpu.MemorySpace.VMEM_SHARED((NPAD, 16), _f32),
            pltpu.MemorySpace.VMEM((4, C), _i32),
            pltpu.MemorySpace.VMEM((4, C), _i32),
            pltpu.MemorySpace.VMEM((2, C, 2), _f32),
            pltpu.MemorySpace.VMEM((2, C, 2), _f32),
            pltpu.MemorySpace.VMEM((2, C, HID), _f32),
            pltpu.MemorySpace.VMEM((2, C, 16), _f32),
            pltpu.SemaphoreType.DMA((4,)),
            pltpu.SemaphoreType.DMA((2,)),
            pltpu.SemaphoreType.DMA((2,)),
        ],
    )
    return f(h2, t2, srcp, dstp, zz)[0]


def _tc_final(a0_ref, a1_ref, ws2_ref, h2_ref, b2_ref, bat_ref,
              f1w_ref, f1b_ref, f2w_ref, f2b_ref, out_ref, accg, acccnt):
    i = pl.program_id(0)
    nsteps = pl.num_programs(0)
    a0 = a0_ref[...]
    a1 = a1_ref[...]
    ws2 = ws2_ref[...]
    num = a0[:, :8] + a1[:, :8] + ws2 * h2_ref[...]
    den = a0[:, 8:9] + a1[:, 8:9] + ws2
    h2f = num / (den + 1e-16) + b2_ref[...]
    bat = bat_ref[0]
    gid = lax.broadcasted_iota(_i32, (NG, 1), 0)
    oh = (bat == gid).astype(_f32)
    part = jnp.dot(oh, h2f, preferred_element_type=_f32)
    cnt = jnp.sum(oh, axis=1, keepdims=True)

    @pl.when(i == 0)
    def _():
        accg[...] = part
        acccnt[...] = cnt

    @pl.when(i > 0)
    def _():
        accg[...] += part
        acccnt[...] += cnt

    @pl.when(i == nsteps - 1)
    def _():
        g = accg[...] / jnp.maximum(acccnt[...], 1.0)
        z = jnp.maximum(jnp.dot(g, f1w_ref[...], preferred_element_type=_f32)
                        + f1b_ref[...], 0.0)
        z2 = jnp.dot(z, f2w_ref[...], preferred_element_type=_f32) + f2b_ref[...]
        mx = jnp.max(z2, axis=1, keepdims=True)
        lse = mx + jnp.log(jnp.sum(jnp.exp(z2 - mx), axis=1, keepdims=True))
        out_ref[...] = z2 - lse


def _run_final(acc0, acc1, wself2, h2, b2r, bat2d, fc1_w, fc1b, fc2_w, fc2b):
    B = 2000
    grid = (N // B,)
    return pl.pallas_call(
        _tc_final,
        grid=grid,
        in_specs=[
            pl.BlockSpec((B, 16), lambda i: (i, 0)),
            pl.BlockSpec((B, 16), lambda i: (i, 0)),
            pl.BlockSpec((B, 1), lambda i: (i, 0)),
            pl.BlockSpec((B, HID), lambda i: (i, 0)),
            pl.BlockSpec((1, HID), lambda i: (0, 0)),
            pl.BlockSpec((1, 1, B), lambda i: (i, 0, 0)),
            pl.BlockSpec((HID, 20), lambda i: (0, 0)),
            pl.BlockSpec((1, 20), lambda i: (0, 0)),
            pl.BlockSpec((20, NC), lambda i: (0, 0)),
            pl.BlockSpec((1, NC), lambda i: (0, 0)),
        ],
        out_specs=pl.BlockSpec((NG, NC), lambda i: (0, 0)),
        out_shape=jax.ShapeDtypeStruct((NG, NC), _f32),
        scratch_shapes=[
            pltpu.MemorySpace.VMEM((NG, HID), _f32),
            pltpu.MemorySpace.VMEM((NG, 1), _f32),
        ],
    )(acc0, acc1, wself2, h2, b2r, bat2d, fc1_w, fc1b, fc2_w, fc2b)


def kernel(x, edge_index, batch, W1, a_src1, a_dst1, b1, W2, a_src2, a_dst2,
           b2, fc1_w, fc1_b, fc2_w, fc2_b):
    E = edge_index.shape[1]
    npd = EP - E
    src_p = jnp.concatenate([edge_index[0], jnp.zeros((npd,), _i32)])
    dst_p = jnp.concatenate(
        [edge_index[1], N + (jnp.arange(npd, dtype=_i32) % 128)])
    srcp = src_p.reshape(NW * CH, C)
    dstp = dst_p.reshape(NW * CH, C)

    rows = np.arange(H1 * HID)
    Msrc = jnp.zeros((H1 * HID, H1), _f32).at[rows, rows // HID].set(
        a_src1.reshape(-1))
    Mdst = jnp.zeros((H1 * HID, H1), _f32).at[rows, rows // HID].set(
        a_dst1.reshape(-1))
    E8 = jnp.asarray((np.arange(64)[None, :] // 8 == np.arange(8)[:, None])
                     .astype(np.float32))
    zz1 = jnp.zeros((NPAD, 72), _f32)
    zz2 = jnp.zeros((NPAD, 16), _f32)

    h, asrc, adst, wself = _run_pre1(x, W1, Msrc, Mdst)
    accp = _run_sc1(h, asrc, adst, srcp, dstp, zz1)
    h2, asrc2, adst2, wself2 = _run_merge1(
        accp[0, :N], accp[1, :N], wself, h, E8, b1.reshape(1, 64), W2,
        a_src2.reshape(HID, 1), a_dst2.reshape(HID, 1))
    t2 = jnp.concatenate([asrc2, adst2], axis=1)
    accp2 = _run_sc2(h2, t2, srcp, dstp, zz2)
    out = _run_final(accp2[0, :N], accp2[1, :N], wself2, h2,
                     b2.reshape(1, HID), batch.reshape(N // 2000, 1, 2000),
                     fc1_w, fc1_b.reshape(1, 20), fc2_w, fc2_b.reshape(1, NC))
    return out


# speedup vs baseline: 169.9287x; 1.6781x over previous
import functools

import jax
import jax.numpy as jnp
import numpy as np
from jax import lax
from jax.experimental import pallas as pl
from jax.experimental.pallas import tpu as pltpu
from jax.experimental.pallas import tpu_sc as plsc

N = 10000
DIN = 128
HID = 8
H1 = 8
NG = 64
NC = 10

NW = 32
C = 80
CH = 128
EPW = C * CH
EP = NW * EPW
NPAD = N + 128

_f32 = jnp.float32
_i32 = jnp.int32


def _dyngather(v, idx):
    dnums = lax.GatherDimensionNumbers(
        offset_dims=(), collapsed_slice_dims=(0,), start_index_map=(0,))
    return lax.gather(v, idx[:, None], dnums, (1,),
                      mode=lax.GatherScatterMode.PROMISE_IN_BOUNDS)


def _tc_pre1(x_ref, w1_ref, ms_ref, md_ref, h_ref, as_ref, ad_ref, ws_ref):
    h = jnp.dot(x_ref[...], w1_ref[...], preferred_element_type=_f32)
    h_ref[...] = h
    a_s = jnp.dot(h, ms_ref[...], preferred_element_type=_f32)
    a_d = jnp.dot(h, md_ref[...], preferred_element_type=_f32)
    as_ref[...] = a_s
    ad_ref[...] = a_d
    s = a_s + a_d
    s = jnp.maximum(s, 0.2 * s)
    ws_ref[...] = jnp.exp(s)


def _run_pre1(x, W1, Msrc, Mdst):
    B = 2000
    grid = (N // B,)
    return pl.pallas_call(
        _tc_pre1,
        grid=grid,
        in_specs=[
            pl.BlockSpec((B, DIN), lambda i: (i, 0)),
            pl.BlockSpec((DIN, H1 * HID), lambda i: (0, 0)),
            pl.BlockSpec((H1 * HID, H1), lambda i: (0, 0)),
            pl.BlockSpec((H1 * HID, H1), lambda i: (0, 0)),
        ],
        out_specs=[
            pl.BlockSpec((B, H1 * HID), lambda i: (i, 0)),
            pl.BlockSpec((B, H1), lambda i: (i, 0)),
            pl.BlockSpec((B, H1), lambda i: (i, 0)),
            pl.BlockSpec((B, H1), lambda i: (i, 0)),
        ],
        out_shape=[
            jax.ShapeDtypeStruct((N, H1 * HID), _f32),
            jax.ShapeDtypeStruct((N, H1), _f32),
            jax.ShapeDtypeStruct((N, H1), _f32),
            jax.ShapeDtypeStruct((N, H1), _f32),
        ],
    )(x, W1, Msrc, Mdst)


def _sc1_body(h_hbm, as_hbm, ad_hbm, src_hbm, dst_hbm, zz_hbm, acc_hbm,
              h_sp, as_sp, ad_sp, acc_sp,
              src_t, dst_t, as_g, ad_g, h_g, buf, sem_i, sem_g, sem_s):
    cid = lax.axis_index("c")
    sid = lax.axis_index("s")
    wid = cid * 16 + sid

    s6 = jnp.minimum(sid * 632, N - 632)
    s3 = jnp.minimum(sid * 640, NPAD - 640)
    pltpu.sync_copy(h_hbm.at[pl.ds(s6, 632)], h_sp.at[pl.ds(s6, 632)])
    pltpu.sync_copy(as_hbm.at[pl.ds(s6, 632)], as_sp.at[pl.ds(s6, 632)])
    pltpu.sync_copy(ad_hbm.at[pl.ds(s6, 632)], ad_sp.at[pl.ds(s6, 632)])
    pltpu.sync_copy(zz_hbm.at[pl.ds(s3, 640)], acc_sp.at[pl.ds(s3, 640)])
    plsc.subcore_barrier()

    lane = lax.iota(_i32, 16)
    ilo8 = lane // 8
    col8 = lane % 8
    idxc = [jnp.where(lane < 8, 2 * j, 2 * j + 1).astype(_i32)
            for j in range(8)]
    base = wid * CH

    def idx_issue(j, s4):
        pltpu.async_copy(src_hbm.at[pl.ds(base + j, 1)],
                         src_t.at[pl.ds(s4, 1)], sem_i.at[s4])
        pltpu.async_copy(dst_hbm.at[pl.ds(base + j, 1)],
                         dst_t.at[pl.ds(s4, 1)], sem_i.at[s4])

    def idx_wait(s4):
        pltpu.make_async_copy(src_hbm.at[pl.ds(0, 1)],
                              src_t.at[pl.ds(s4, 1)], sem_i.at[s4]).wait()
        pltpu.make_async_copy(dst_hbm.at[pl.ds(0, 1)],
                              dst_t.at[pl.ds(s4, 1)], sem_i.at[s4]).wait()

    def gat_issue(s4, s2):
        src_row = src_t.at[s4]
        dst_row = dst_t.at[s4]
        pltpu.async_copy(as_sp.at[src_row], as_g.at[s2], sem_g.at[s2])
        pltpu.async_copy(ad_sp.at[dst_row], ad_g.at[s2], sem_g.at[s2])
        pltpu.async_copy(h_sp.at[src_row], h_g.at[s2], sem_g.at[s2])

    def gat_wait(s4, s2):
        src_row = src_t.at[s4]
        dst_row = dst_t.at[s4]
        pltpu.make_async_copy(as_sp.at[src_row], as_g.at[s2], sem_g.at[s2]).wait()
        pltpu.make_async_copy(ad_sp.at[dst_row], ad_g.at[s2], sem_g.at[s2]).wait()
        pltpu.make_async_copy(h_sp.at[src_row], h_g.at[s2], sem_g.at[s2]).wait()

    def sc_issue(s4, s2):
        pltpu.async_copy(buf.at[s2], acc_sp.at[dst_t.at[s4]], sem_s.at[s2],
                         add=True)

    def sc_wait(s4, s2):
        pltpu.make_async_copy(buf.at[s2], acc_sp.at[dst_t.at[s4]],
                              sem_s.at[s2]).wait()

    idx_issue(0, 0)
    idx_issue(1, 1)
    idx_wait(0)
    gat_issue(0, 0)

    def chunk_body(j, carry):
        s2 = lax.rem(j, 2)
        s2n = 1 - s2
        s4 = lax.rem(j, 4)
        gat_wait(s4, s2)

        @pl.when(j >= 2)
        def _():
            sc_wait(lax.rem(j - 2, 4), s2)

        @pl.when(j < CH - 1)
        def _():
            idx_wait(lax.rem(j + 1, 4))
            gat_issue(lax.rem(j + 1, 4), s2n)

        @pl.when(j < CH - 2)
        def _():
            idx_issue(j + 2, lax.rem(j + 2, 4))

        sl16 = jnp.broadcast_to(s2, (16,)).astype(_i32)

        @functools.partial(plsc.parallel_loop, 0, C // 2, unroll=4)
        def pair_body(p):
            rows2 = 2 * p + ilo8
            a = plsc.load_gather(as_g, [sl16, rows2, col8])
            b = plsc.load_gather(ad_g, [sl16, rows2, col8])
            s = a + b
            s = jnp.maximum(s, 0.2 * s)
            w16 = jnp.exp(s)
            plsc.store_scatter(buf, [sl16, rows2, 64 + col8], w16)
            er0 = jnp.broadcast_to(2 * p, (16,)).astype(_i32)
            er1 = er0 + 1
            for j8 in range(8):
                er = er1 if j8 >= 4 else er0
                cols = (j8 % 4) * 16 + lane
                hv = plsc.load_gather(h_g, [sl16, er, cols])
                wb = _dyngather(w16, idxc[j8])
                plsc.store_scatter(buf, [sl16, er, cols], wb * hv)

        sc_issue(s4, s2)
        return carry

    lax.fori_loop(0, CH, chunk_body, 0)
    sc_wait((CH - 2) % 4, (CH - 2) % 2)
    sc_wait((CH - 1) % 4, (CH - 1) % 2)
    plsc.subcore_barrier()
    pltpu.sync_copy(acc_sp.at[pl.ds(s3, 640)],
                    acc_hbm.at[cid].at[pl.ds(s3, 640)])


def _run_sc1(h, asrc, adst, srcp, dstp, zz):
    mesh = plsc.VectorSubcoreMesh(core_axis_name="c", subcore_axis_name="s", num_cores=2, num_subcores=16)
    f = pl.kernel(
        _sc1_body,
        out_type=[jax.ShapeDtypeStruct((2, NPAD, 72), _f32)],
        mesh=mesh,
        compiler_params=pltpu.CompilerParams(needs_layout_passes=False, use_tc_tiling_on_sc=False),
        scratch_types=[
            pltpu.MemorySpace.VMEM_SHARED((N, 64), _f32),
            pltpu.MemorySpace.VMEM_SHARED((N, 8), _f32),
            pltpu.MemorySpace.VMEM_SHARED((N, 8), _f32),
            pltpu.MemorySpace.VMEM_SHARED((NPAD, 72), _f32),
            pltpu.MemorySpace.VMEM((4, C), _i32),
            pltpu.MemorySpace.VMEM((4, C), _i32),
            pltpu.MemorySpace.VMEM((2, C, 8), _f32),
            pltpu.MemorySpace.VMEM((2, C, 8), _f32),
            pltpu.MemorySpace.VMEM((2, C, 64), _f32),
            pltpu.MemorySpace.VMEM((2, C, 72), _f32),
            pltpu.SemaphoreType.DMA((4,)),
            pltpu.SemaphoreType.DMA((2,)),
            pltpu.SemaphoreType.DMA((2,)),
        ],
    )
    return f(h, asrc, adst, srcp, dstp, zz)[0]


def _tc_merge1(a0_ref, a1_ref, ws_ref, h_ref, e8_ref, b1_ref, w2_ref,
               ms2_ref, md2_ref, h2_ref, as2_ref, ad2_ref, ws2_ref):
    a0 = a0_ref[...]
    a1 = a1_ref[...]
    ws = ws_ref[...]
    h = h_ref[...]
    e8 = e8_ref[...]
    out_t = a0[:, :64] + a1[:, :64] + jnp.dot(ws, e8, preferred_element_type=_f32) * h
    ssum = a0[:, 64:72] + a1[:, 64:72] + ws
    inv = 1.0 / (ssum + 1e-16)
    x2 = out_t * jnp.dot(inv, e8, preferred_element_type=_f32) + b1_ref[...]
    x2 = jnp.where(x2 > 0, x2, jnp.exp(x2) - 1.0)
    h2 = jnp.dot(x2, w2_ref[...], preferred_element_type=_f32)
    h2_ref[...] = h2
    a_s = jnp.dot(h2, ms2_ref[...], preferred_element_type=_f32)
    a_d = jnp.dot(h2, md2_ref[...], preferred_element_type=_f32)
    as2_ref[...] = a_s
    ad2_ref[...] = a_d
    s = a_s + a_d
    s = jnp.maximum(s, 0.2 * s)
    ws2_ref[...] = jnp.exp(s)


def _run_merge1(acc0, acc1, wself, h, E8, b1r, W2, ms2, md2):
    B = 2000
    grid = (N // B,)
    return pl.pallas_call(
        _tc_merge1,
        grid=grid,
        in_specs=[
            pl.BlockSpec((B, 72), lambda i: (i, 0)),
            pl.BlockSpec((B, 72), lambda i: (i, 0)),
            pl.BlockSpec((B, H1), lambda i: (i, 0)),
            pl.BlockSpec((B, 64), lambda i: (i, 0)),
            pl.BlockSpec((H1, 64), lambda i: (0, 0)),
            pl.BlockSpec((1, 64), lambda i: (0, 0)),
            pl.BlockSpec((64, HID), lambda i: (0, 0)),
            pl.BlockSpec((HID, 1), lambda i: (0, 0)),
            pl.BlockSpec((HID, 1), lambda i: (0, 0)),
        ],
        out_specs=[
            pl.BlockSpec((B, HID), lambda i: (i, 0)),
            pl.BlockSpec((B, 1), lambda i: (i, 0)),
            pl.BlockSpec((B, 1), lambda i: (i, 0)),
            pl.BlockSpec((B, 1), lambda i: (i, 0)),
        ],
        out_shape=[
            jax.ShapeDtypeStruct((N, HID), _f32),
            jax.ShapeDtypeStruct((N, 1), _f32),
            jax.ShapeDtypeStruct((N, 1), _f32),
            jax.ShapeDtypeStruct((N, 1), _f32),
        ],
    )(acc0, acc1, wself, h, E8, b1r, W2, ms2, md2)


def _sc2_body(h2_hbm, t2_hbm, src_hbm, dst_hbm, zz_hbm, acc_hbm,
              h2_sp, t2_sp, acc_sp,
              src_t, dst_t, ta_g, tb_g, h2_g, buf, sem_i, sem_g, sem_s):
    cid = lax.axis_index("c")
    sid = lax.axis_index("s")
    wid = cid * 16 + sid

    s6 = jnp.minimum(sid * 632, N - 632)
    s3 = jnp.minimum(sid * 640, NPAD - 640)
    pltpu.sync_copy(h2_hbm.at[pl.ds(s6, 632)], h2_sp.at[pl.ds(s6, 632)])
    pltpu.sync_copy(t2_hbm.at[pl.ds(s6, 632)], t2_sp.at[pl.ds(s6, 632)])
    pltpu.sync_copy(zz_hbm.at[pl.ds(s3, 640)], acc_sp.at[pl.ds(s3, 640)])
    plsc.subcore_barrier()

    lane = lax.iota(_i32, 16)
    col8 = lane % 8
    zz16 = jnp.zeros((16,), _i32)
    on16 = jnp.ones((16,), _i32)
    idxb = [jnp.full((16,), j, _i32) for j in range(16)]
    base = wid * CH

    def idx_issue(j, s4):
        pltpu.async_copy(src_hbm.at[pl.ds(base + j, 1)],
                         src_t.at[pl.ds(s4, 1)], sem_i.at[s4])
        pltpu.async_copy(dst_hbm.at[pl.ds(base + j, 1)],
                         dst_t.at[pl.ds(s4, 1)], sem_i.at[s4])

    def idx_wait(s4):
        pltpu.make_async_copy(src_hbm.at[pl.ds(0, 1)],
                              src_t.at[pl.ds(s4, 1)], sem_i.at[s4]).wait()
        pltpu.make_async_copy(dst_hbm.at[pl.ds(0, 1)],
                              dst_t.at[pl.ds(s4, 1)], sem_i.at[s4]).wait()

    def gat_issue(s4, s2):
        src_row = src_t.at[s4]
        dst_row = dst_t.at[s4]
        pltpu.async_copy(t2_sp.at[src_row], ta_g.at[s2], sem_g.at[s2])
        pltpu.async_copy(t2_sp.at[dst_row], tb_g.at[s2], sem_g.at[s2])
        pltpu.async_copy(h2_sp.at[src_row], h2_g.at[s2], sem_g.at[s2])

    def gat_wait(s4, s2):
        src_row = src_t.at[s4]
        dst_row = dst_t.at[s4]
        pltpu.make_async_copy(t2_sp.at[src_row], ta_g.at[s2], sem_g.at[s2]).wait()
        pltpu.make_async_copy(t2_sp.at[dst_row], tb_g.at[s2], sem_g.at[s2]).wait()
        pltpu.make_async_copy(h2_sp.at[src_row], h2_g.at[s2], sem_g.at[s2]).wait()

    def sc_issue(s4, s2):
        pltpu.async_copy(buf.at[s2], acc_sp.at[dst_t.at[s4]], sem_s.at[s2],
                         add=True)

    def sc_wait(s4, s2):
        pltpu.make_async_copy(buf.at[s2], acc_sp.at[dst_t.at[s4]],
                              sem_s.at[s2]).wait()

    idx_issue(0, 0)
    idx_issue(1, 1)
    idx_wait(0)
    gat_issue(0, 0)

    def chunk_body(j, carry):
        s2 = lax.rem(j, 2)
        s2n = 1 - s2
        s4 = lax.rem(j, 4)
        gat_wait(s4, s2)

        @pl.when(j >= 2)
        def _():
            sc_wait(lax.rem(j - 2, 4), s2)

        @pl.when(j < CH - 1)
        def _():
            idx_wait(lax.rem(j + 1, 4))
            gat_issue(lax.rem(j + 1, 4), s2n)

        @pl.when(j < CH - 2)
        def _():
            idx_issue(j + 2, lax.rem(j + 2, 4))

        sl16 = jnp.broadcast_to(s2, (16,)).astype(_i32)

        @functools.partial(plsc.parallel_loop, 0, C // 16, unroll=2)
        def grp_body(g):
            rows16 = g * 16 + lane
            a = plsc.load_gather(ta_g, [sl16, rows16, zz16])
            b = plsc.load_gather(tb_g, [sl16, rows16, on16])
            s = a + b
            s = jnp.maximum(s, 0.2 * s)
            w16 = jnp.exp(s)
            for j16 in range(16):
                e_rel = g * 16 + j16
                er = jnp.broadcast_to(e_rel, (16,)).astype(_i32)
                hv = plsc.load_gather(h2_g, [sl16, er, col8])
                wb = _dyngather(w16, idxb[j16])
                m = jnp.where(lane < 8, wb * hv,
                              jnp.where(lane == 8, wb, 0.0))
                plsc.store_scatter(buf, [sl16, er, lane], m)

        sc_issue(s4, s2)
        return carry

    lax.fori_loop(0, CH, chunk_body, 0)
    sc_wait((CH - 2) % 4, (CH - 2) % 2)
    sc_wait((CH - 1) % 4, (CH - 1) % 2)
    plsc.subcore_barrier()
    pltpu.sync_copy(acc_sp.at[pl.ds(s3, 640)],
                    acc_hbm.at[cid].at[pl.ds(s3, 640)])


def _run_sc2(h2, t2, srcp, dstp, zz):
    mesh = plsc.VectorSubcoreMesh(core_axis_name="c", subcore_axis_name="s", num_cores=2, num_subcores=16)
    f = pl.kernel(
        _sc2_body,
        out_type=[jax.ShapeDtypeStruct((2, NPAD, 16), _f32)],
        mesh=mesh,
        compiler_params=pltpu.CompilerParams(needs_layout_passes=False, use_tc_tiling_on_sc=False),
        scratch_types=[
            pltpu.MemorySpace.VMEM_SHARED((N, HID), _f32),
            pltpu.MemorySpace.VMEM_SHARED((N, 2), _f32),
            pltpu.MemorySpace.VMEM_SHARED((NPAD, 16), _f32),
            pltpu.MemorySpace.VMEM((4, C), _i32),
            pltpu.MemorySpace.VMEM((4, C), _i32),
            pltpu.MemorySpace.VMEM((2, C, 2), _f32),
            pltpu.MemorySpace.VMEM((2, C, 2), _f32),
            pltpu.MemorySpace.VMEM((2, C, HID), _f32),
            pltpu.MemorySpace.VMEM((2, C, 16), _f32),
            pltpu.SemaphoreType.DMA((4,)),
            pltpu.SemaphoreType.DMA((2,)),
            pltpu.SemaphoreType.DMA((2,)),
        ],
    )
    return f(h2, t2, srcp, dstp, zz)[0]


def _tc_final(a0_ref, a1_ref, ws2_ref, h2_ref, b2_ref, bat_ref,
              f1w_ref, f1b_ref, f2w_ref, f2b_ref, out_ref, accg, acccnt):
    i = pl.program_id(0)
    nsteps = pl.num_programs(0)
    a0 = a0_ref[...]
    a1 = a1_ref[...]
    ws2 = ws2_ref[...]
    num = a0[:, :8] + a1[:, :8] + ws2 * h2_ref[...]
    den = a0[:, 8:9] + a1[:, 8:9] + ws2
    h2f = num / (den + 1e-16) + b2_ref[...]
    bat = bat_ref[0]
    gid = lax.broadcasted_iota(_i32, (NG, 1), 0)
    oh = (bat == gid).astype(_f32)
    part = jnp.dot(oh, h2f, preferred_element_type=_f32)
    cnt = jnp.sum(oh, axis=1, keepdims=True)

    @pl.when(i == 0)
    def _():
        accg[...] = part
        acccnt[...] = cnt

    @pl.when(i > 0)
    def _():
        accg[...] += part
        acccnt[...] += cnt

    @pl.when(i == nsteps - 1)
    def _():
        g = accg[...] / jnp.maximum(acccnt[...], 1.0)
        z = jnp.maximum(jnp.dot(g, f1w_ref[...], preferred_element_type=_f32)
                        + f1b_ref[...], 0.0)
        z2 = jnp.dot(z, f2w_ref[...], preferred_element_type=_f32) + f2b_ref[...]
        mx = jnp.max(z2, axis=1, keepdims=True)
        lse = mx + jnp.log(jnp.sum(jnp.exp(z2 - mx), axis=1, keepdims=True))
        out_ref[...] = z2 - lse


def _run_final(acc0, acc1, wself2, h2, b2r, bat2d, fc1_w, fc1b, fc2_w, fc2b):
    B = 2000
    grid = (N // B,)
    return pl.pallas_call(
        _tc_final,
        grid=grid,
        in_specs=[
            pl.BlockSpec((B, 16), lambda i: (i, 0)),
            pl.BlockSpec((B, 16), lambda i: (i, 0)),
            pl.BlockSpec((B, 1), lambda i: (i, 0)),
            pl.BlockSpec((B, HID), lambda i: (i, 0)),
            pl.BlockSpec((1, HID), lambda i: (0, 0)),
            pl.BlockSpec((1, 1, B), lambda i: (i, 0, 0)),
            pl.BlockSpec((HID, 20), lambda i: (0, 0)),
            pl.BlockSpec((1, 20), lambda i: (0, 0)),
            pl.BlockSpec((20, NC), lambda i: (0, 0)),
            pl.BlockSpec((1, NC), lambda i: (0, 0)),
        ],
        out_specs=pl.BlockSpec((NG, NC), lambda i: (0, 0)),
        out_shape=jax.ShapeDtypeStruct((NG, NC), _f32),
        scratch_shapes=[
            pltpu.MemorySpace.VMEM((NG, HID), _f32),
            pltpu.MemorySpace.VMEM((NG, 1), _f32),
        ],
    )(acc0, acc1, wself2, h2, b2r, bat2d, fc1_w, fc1b, fc2_w, fc2b)


def kernel(x, edge_index, batch, W1, a_src1, a_dst1, b1, W2, a_src2, a_dst2,
           b2, fc1_w, fc1_b, fc2_w, fc2_b):
    E = edge_index.shape[1]
    npd = EP - E
    src_p = jnp.concatenate([edge_index[0], jnp.zeros((npd,), _i32)])
    dst_p = jnp.concatenate(
        [edge_index[1], N + (jnp.arange(npd, dtype=_i32) % 128)])
    srcp = src_p.reshape(NW * CH, C)
    dstp = dst_p.reshape(NW * CH, C)

    rows = np.arange(H1 * HID)
    Msrc = jnp.zeros((H1 * HID, H1), _f32).at[rows, rows // HID].set(
        a_src1.reshape(-1))
    Mdst = jnp.zeros((H1 * HID, H1), _f32).at[rows, rows // HID].set(
        a_dst1.reshape(-1))
    E8 = jnp.asarray((np.arange(64)[None, :] // 8 == np.arange(8)[:, None])
                     .astype(np.float32))
    zz1 = jnp.zeros((NPAD, 72), _f32)
    zz2 = jnp.zeros((NPAD, 16), _f32)

    h, asrc, adst, wself = _run_pre1(x, W1, Msrc, Mdst)
    accp = _run_sc1(h, asrc, adst, srcp, dstp, zz1)
    h2, asrc2, adst2, wself2 = _run_merge1(
        accp[0, :N], accp[1, :N], wself, h, E8, b1.reshape(1, 64), W2,
        a_src2.reshape(HID, 1), a_dst2.reshape(HID, 1))
    t2 = jnp.concatenate([asrc2, adst2], axis=1)
    accp2 = _run_sc2(h2, t2, srcp, dstp, zz2)
    out = _run_final(accp2[0, :N], accp2[1, :N], wself2, h2,
                     b2.reshape(1, HID), batch.reshape(N // 2000, 1, 2000),
                     fc1_w, fc1_b.reshape(1, 20), fc2_w, fc2_b.reshape(1, NC))
    return out
